# baseline (speedup 1.0000x reference)
import functools

import jax
import jax.numpy as jnp
from jax import lax
from jax.experimental import pallas as pl
from jax.experimental.pallas import tpu as pltpu
from jax.experimental.pallas import tpu_sc as plsc

_LANES = 16
_NBUF = 4


@functools.cache
def _build_kernel(B, S, D, V):
    info = plsc.get_sparse_core_info()
    NW = info.num_cores * info.num_subcores
    BB = B // NW
    DT = D // 8
    TILE = 8 * 128
    assert BB == 128 and S % _NBUF == 0

    mesh = plsc.VectorSubcoreMesh(core_axis_name="c", subcore_axis_name="s")

    @functools.partial(
        pl.kernel,
        out_type=jax.ShapeDtypeStruct((S, DT, NW, TILE), jnp.float32),
        mesh=mesh,
        scratch_types=(
            [pltpu.VMEM((BB * S,), jnp.int32)]
            + [pltpu.VMEM((S, D), jnp.float32)]
            + [pltpu.VMEM((BB,), jnp.int32) for _ in range(_NBUF)]
            + [pltpu.VMEM((BB, D), jnp.float32) for _ in range(_NBUF)]
            + [pltpu.VMEM((DT * TILE,), jnp.float32) for _ in range(_NBUF)]
            + [pltpu.VMEM((16 * _LANES,), jnp.int32)]
            + [pltpu.VMEM((16 * _LANES,), jnp.int32)]
            + [pltpu.SemaphoreType.DMA for _ in range(2 * _NBUF)]
        ),
        compiler_params=pltpu.CompilerParams(
            use_tc_tiling_on_sc=False, needs_layout_passes=False,
            disable_bounds_checks=True),
    )
    def embed(idx_hbm, table_hbm, pos_hbm, out_hbm, idx_blk, pos_v, *bufs):
        icol = bufs[:_NBUF]
        rows = bufs[_NBUF:2 * _NBUF]
        outb = bufs[2 * _NBUF:3 * _NBUF]
        btab = bufs[3 * _NBUF]
        otab = bufs[3 * _NBUF + 1]
        gsem = bufs[3 * _NBUF + 2:4 * _NBUF + 2]
        wsem = bufs[4 * _NBUF + 2:]
        wid = lax.axis_index("s") * info.num_cores + lax.axis_index("c")

        pltpu.sync_copy(idx_hbm.at[pl.ds(wid * (BB * S), BB * S)], idx_blk)
        pltpu.sync_copy(pos_hbm, pos_v)

        lane = lax.iota(jnp.int32, _LANES)
        lane_s = lane * S
        oconst = (lane >> 3) * TILE + (lane & 7) * 128

        def tab_body(k, carry):
            t = (lane + k) & 15
            btab[pl.ds(k * _LANES, _LANES)] = t
            otab[pl.ds(k * _LANES, _LANES)] = oconst + t
            return carry

        lax.fori_loop(0, 16, tab_body, 0)

        def stage_gather(s, j):
            for bb in range(BB // _LANES):
                v = plsc.load_gather(
                    idx_blk, [lane_s + (bb * _LANES * S + s)])
                icol[j][pl.ds(bb * _LANES, _LANES)] = v
            pltpu.make_async_copy(
                table_hbm.at[icol[j]], rows[j], gsem[j]).start()

        def write_desc(s, tr, j):
            return pltpu.make_async_copy(
                outb[j].at[pl.ds(tr * TILE, TILE)],
                out_hbm.at[s, tr, wid], wsem[j])

        for j in range(_NBUF - 1):
            stage_gather(j, j)

        def block_body(blk, carry):
            for b in range(_NBUF):
                s = blk * _NBUF + b
                jprev = (b - 1) % _NBUF
                jnext = (b + _NBUF - 1) % _NBUF
                if b == 0:
                    @pl.when(blk >= 1)
                    def _():
                        for tr in range(DT):
                            write_desc(s - 1, tr, jprev).wait()
                else:
                    for tr in range(DT):
                        write_desc(s - 1, tr, jprev).wait()
                @pl.when(s + _NBUF - 1 < S)
                def _():
                    stage_gather(s + _NBUF - 1, jnext)
                pltpu.make_async_copy(
                    table_hbm.at[icol[b]], rows[b], gsem[b]).wait()

                @plsc.parallel_loop(0, 16, unroll=4)
                def _(k):
                    bt = btab[pl.ds(k * _LANES, _LANES)]
                    ot = otab[pl.ds(k * _LANES, _LANES)]
                    for db in range(D // _LANES):
                        pv = pos_v[s, pl.ds(db * _LANES, _LANES)]
                        dv = lane + db * _LANES
                        for bb in range(BB // _LANES):
                            v = plsc.load_gather(
                                rows[b], [bt + bb * _LANES, dv])
                            plsc.store_scatter(
                                outb[b],
                                [ot + (db * 2 * TILE + bb * _LANES)],
                                v + pv)
                for tr in range(DT):
                    write_desc(s, tr, b).start()
            return carry

        lax.fori_loop(0, S // _NBUF, block_body, 0)
        for tr in range(DT):
            write_desc(S - 1, tr, (S - 1) % _NBUF).wait()

    return embed


def kernel(inputs, token_table, position_table):
    B, S = inputs.shape
    V, D = token_table.shape
    NW = 32
    idx_flat = inputs.reshape(B * S).astype(jnp.int32)
    fn = _build_kernel(B, S, D, V)
    out = fn(idx_flat, token_table, position_table)
    out = out.reshape(S, D // 8, NW, 8, 128)
    out = out.transpose(2, 4, 0, 1, 3)
    return out.reshape(B, S, D)


# speedup vs baseline: 1.5300x; 1.5300x over previous
import functools

import jax
import jax.numpy as jnp
from jax import lax
from jax.experimental import pallas as pl
from jax.experimental.pallas import tpu as pltpu
from jax.experimental.pallas import tpu_sc as plsc

_LANES = 16
_NBUF = 4


@functools.cache
def _build_kernel(B, S, D, V):
    info = plsc.get_sparse_core_info()
    NW = info.num_cores * info.num_subcores
    BB = B // NW
    DT = D // 8
    TILE = 8 * 128
    assert BB == 128 and S % _NBUF == 0

    mesh = plsc.VectorSubcoreMesh(core_axis_name="c", subcore_axis_name="s")

    @functools.partial(
        pl.kernel,
        out_type=jax.ShapeDtypeStruct((S, DT, NW, TILE), jnp.float32),
        mesh=mesh,
        scratch_types=(
            [pltpu.VMEM((BB * S,), jnp.int32)]
            + [pltpu.VMEM((S, D), jnp.float32)]
            + [pltpu.VMEM((BB,), jnp.int32) for _ in range(_NBUF)]
            + [pltpu.VMEM((BB, D), jnp.float32) for _ in range(_NBUF)]
            + [pltpu.VMEM((DT * TILE,), jnp.float32) for _ in range(_NBUF)]
            + [pltpu.VMEM((16 * _LANES,), jnp.int32)]
            + [pltpu.VMEM((16 * _LANES,), jnp.int32)]
            + [pltpu.SemaphoreType.DMA for _ in range(2 * _NBUF)]
        ),
        compiler_params=pltpu.CompilerParams(
            use_tc_tiling_on_sc=False, needs_layout_passes=False,
            disable_bounds_checks=True),
    )
    def embed(idx_hbm, table_hbm, pos_hbm, out_hbm, idx_blk, pos_v, *bufs):
        icol = bufs[:_NBUF]
        rows = bufs[_NBUF:2 * _NBUF]
        outb = bufs[2 * _NBUF:3 * _NBUF]
        btab = bufs[3 * _NBUF]
        otab = bufs[3 * _NBUF + 1]
        gsem = bufs[3 * _NBUF + 2:4 * _NBUF + 2]
        wsem = bufs[4 * _NBUF + 2:]
        wid = lax.axis_index("s") * info.num_cores + lax.axis_index("c")

        pltpu.sync_copy(idx_hbm.at[pl.ds(wid * (BB * S), BB * S)], idx_blk)
        pltpu.sync_copy(pos_hbm, pos_v)

        lane = lax.iota(jnp.int32, _LANES)
        lane_s = lane * S
        oconst = (lane >> 3) * TILE + (lane & 7) * 128

        def tab_body(k, carry):
            t = (lane + k) & 15
            btab[pl.ds(k * _LANES, _LANES)] = t
            otab[pl.ds(k * _LANES, _LANES)] = oconst + t
            return carry

        lax.fori_loop(0, 16, tab_body, 0)

        def stage_gather(s, j):
            for bb in range(BB // _LANES):
                v = plsc.load_gather(
                    idx_blk, [lane_s + (bb * _LANES * S + s)])
                icol[j][pl.ds(bb * _LANES, _LANES)] = v
            pltpu.make_async_copy(
                table_hbm.at[icol[j]], rows[j], gsem[j]).start()

        def write_desc(s, tr, j):
            return pltpu.make_async_copy(
                outb[j].at[pl.ds(tr * TILE, TILE)],
                out_hbm.at[s, tr, wid], wsem[j])

        for j in range(_NBUF - 1):
            stage_gather(j, j)

        def block_body(blk, carry):
            for b in range(_NBUF):
                s = blk * _NBUF + b
                jprev = (b - 1) % _NBUF
                jnext = (b + _NBUF - 1) % _NBUF
                if b == 0:
                    @pl.when(blk >= 1)
                    def _():
                        for tr in range(DT):
                            write_desc(s - 1, tr, jprev).wait()
                else:
                    for tr in range(DT):
                        write_desc(s - 1, tr, jprev).wait()
                @pl.when(s + _NBUF - 1 < S)
                def _():
                    stage_gather(s + _NBUF - 1, jnext)
                pltpu.make_async_copy(
                    table_hbm.at[icol[b]], rows[b], gsem[b]).wait()

                @plsc.parallel_loop(0, 16, unroll=2)
                def _(k):
                    bt = btab[pl.ds(k * _LANES, _LANES)]
                    ot = otab[pl.ds(k * _LANES, _LANES)]
                    for db in range(D // _LANES):
                        pv = pos_v[s, pl.ds(db * _LANES, _LANES)]
                        dv = lane + db * _LANES
                        for bb in range(BB // _LANES):
                            base_w = db * 2 * TILE + bb * _LANES
                            v = plsc.load_gather(
                                rows[b].at[pl.ds(bb * _LANES, _LANES)],
                                [bt, dv])
                            plsc.store_scatter(
                                outb[b].at[pl.ds(base_w, DT * TILE - base_w)],
                                [ot], v + pv)
                for tr in range(DT):
                    write_desc(s, tr, b).start()
            return carry

        lax.fori_loop(0, S // _NBUF, block_body, 0)
        for tr in range(DT):
            write_desc(S - 1, tr, (S - 1) % _NBUF).wait()

    return embed


def kernel(inputs, token_table, position_table):
    B, S = inputs.shape
    V, D = token_table.shape
    NW = 32
    idx_flat = inputs.reshape(B * S).astype(jnp.int32)
    fn = _build_kernel(B, S, D, V)
    out = fn(idx_flat, token_table, position_table)
    out = out.reshape(S, D // 8, NW, 8, 128)
    out = out.transpose(2, 4, 0, 1, 3)
    return out.reshape(B, S, D)


# speedup vs baseline: 1.7470x; 1.1418x over previous
import functools

import jax
import jax.numpy as jnp
from jax import lax
from jax.experimental import pallas as pl
from jax.experimental.pallas import tpu as pltpu
from jax.experimental.pallas import tpu_sc as plsc

_LANES = 16
_NBUF = 4


@functools.cache
def _build_kernel(B, S, D, V):
    info = plsc.get_sparse_core_info()
    NW = info.num_cores * info.num_subcores
    BB = B // NW
    DT = D // 8
    TILE = 8 * 128
    assert BB == 128 and S % _NBUF == 0

    mesh = plsc.VectorSubcoreMesh(core_axis_name="c", subcore_axis_name="s")

    @functools.partial(
        pl.kernel,
        out_type=jax.ShapeDtypeStruct((S, DT, NW, TILE), jnp.float32),
        mesh=mesh,
        scratch_types=(
            [pltpu.VMEM((BB * S,), jnp.int32)]
            + [pltpu.VMEM((S, D), jnp.float32)]
            + [pltpu.VMEM((BB,), jnp.int32) for _ in range(_NBUF)]
            + [pltpu.VMEM((BB, D), jnp.float32) for _ in range(_NBUF)]
            + [pltpu.VMEM((DT * TILE,), jnp.float32) for _ in range(_NBUF)]
            + [pltpu.VMEM((16 * _LANES,), jnp.int32)]
            + [pltpu.VMEM((16 * _LANES,), jnp.int32)]
            + [pltpu.SemaphoreType.DMA for _ in range(2 * _NBUF)]
        ),
        compiler_params=pltpu.CompilerParams(
            use_tc_tiling_on_sc=False, needs_layout_passes=False,
            disable_bounds_checks=True, disable_semaphore_checks=True),
    )
    def embed(idx_hbm, table_hbm, pos_hbm, out_hbm, idx_blk, pos_v, *bufs):
        icol = bufs[:_NBUF]
        rows = bufs[_NBUF:2 * _NBUF]
        outb = bufs[2 * _NBUF:3 * _NBUF]
        btab = bufs[3 * _NBUF]
        otab = bufs[3 * _NBUF + 1]
        gsem = bufs[3 * _NBUF + 2:4 * _NBUF + 2]
        wsem = bufs[4 * _NBUF + 2:]
        wid = lax.axis_index("s") * info.num_cores + lax.axis_index("c")

        pltpu.sync_copy(idx_hbm.at[pl.ds(wid * (BB * S), BB * S)], idx_blk)
        pltpu.sync_copy(pos_hbm, pos_v)

        lane = lax.iota(jnp.int32, _LANES)
        lane_s = lane * S
        oconst = (lane >> 3) * TILE + (lane & 7) * 128

        def tab_body(k, carry):
            t = (lane + k) & 15
            btab[pl.ds(k * _LANES, _LANES)] = t
            otab[pl.ds(k * _LANES, _LANES)] = oconst + t
            return carry

        lax.fori_loop(0, 16, tab_body, 0)

        def stage_gather(s, j):
            for bb in range(BB // _LANES):
                v = plsc.load_gather(
                    idx_blk, [lane_s + (bb * _LANES * S + s)])
                icol[j][pl.ds(bb * _LANES, _LANES)] = v
            pltpu.make_async_copy(
                table_hbm.at[icol[j]], rows[j], gsem[j]).start()

        def write_desc(s, tr, j):
            return pltpu.make_async_copy(
                outb[j].at[pl.ds(tr * TILE, TILE)],
                out_hbm.at[s, tr, wid], wsem[j])

        for j in range(_NBUF - 1):
            stage_gather(j, j)

        def block_body(blk, carry):
            for b in range(_NBUF):
                s = blk * _NBUF + b
                jprev = (b - 1) % _NBUF
                jnext = (b + _NBUF - 1) % _NBUF
                if b == 0:
                    @pl.when(blk >= 1)
                    def _():
                        for tr in range(DT):
                            write_desc(s - 1, tr, jprev).wait()
                else:
                    for tr in range(DT):
                        write_desc(s - 1, tr, jprev).wait()
                @pl.when(s + _NBUF - 1 < S)
                def _():
                    stage_gather(s + _NBUF - 1, jnext)
                pltpu.make_async_copy(
                    table_hbm.at[icol[b]], rows[b], gsem[b]).wait()

                @plsc.parallel_loop(0, 64, unroll=4)
                def _(i):
                    k = i >> 2
                    db = i & 3
                    bt = btab[pl.ds(k * _LANES, _LANES)]
                    ot = otab[pl.ds(k * _LANES, _LANES)]
                    if True:
                        pv = pos_v[s, pl.ds(db * _LANES, _LANES)]
                        dv = lane + db * _LANES
                        for bb in range(BB // _LANES):
                            base_w = db * 2 * TILE + bb * _LANES
                            v = plsc.load_gather(
                                rows[b].at[pl.ds(bb * _LANES, _LANES)],
                                [bt, dv])
                            plsc.store_scatter(
                                outb[b].at[pl.ds(base_w, 1936)],
                                [ot], v + pv)
                for tr in range(DT):
                    write_desc(s, tr, b).start()
            return carry

        lax.fori_loop(0, S // _NBUF, block_body, 0)
        for tr in range(DT):
            write_desc(S - 1, tr, (S - 1) % _NBUF).wait()

    return embed


def kernel(inputs, token_table, position_table):
    B, S = inputs.shape
    V, D = token_table.shape
    NW = 32
    idx_flat = inputs.reshape(B * S).astype(jnp.int32)
    fn = _build_kernel(B, S, D, V)
    out = fn(idx_flat, token_table, position_table)
    out = out.reshape(S, D // 8, NW, 8, 128)
    out = out.transpose(2, 4, 0, 1, 3)
    return out.reshape(B, S, D)


# speedup vs baseline: 1.7605x; 1.0077x over previous
import functools

import jax
import jax.numpy as jnp
from jax import lax
from jax.experimental import pallas as pl
from jax.experimental.pallas import tpu as pltpu
from jax.experimental.pallas import tpu_sc as plsc

_LANES = 16
_NBUF = 4


@functools.cache
def _build_kernel(B, S, D, V):
    info = plsc.get_sparse_core_info()
    NW = info.num_cores * info.num_subcores
    BB = B // NW
    DT = D // 8
    TILE = 8 * 128
    assert BB == 128 and S % _NBUF == 0

    mesh = plsc.VectorSubcoreMesh(core_axis_name="c", subcore_axis_name="s")

    @functools.partial(
        pl.kernel,
        out_type=jax.ShapeDtypeStruct((S, DT, NW, TILE), jnp.float32),
        mesh=mesh,
        scratch_types=(
            [pltpu.VMEM((BB * S,), jnp.int32)]
            + [pltpu.VMEM((S, D), jnp.float32)]
            + [pltpu.VMEM((BB,), jnp.int32) for _ in range(_NBUF)]
            + [pltpu.VMEM((BB, D), jnp.float32) for _ in range(_NBUF)]
            + [pltpu.VMEM((DT * TILE,), jnp.float32) for _ in range(_NBUF)]
            + [pltpu.VMEM((16 * _LANES,), jnp.int32)]
            + [pltpu.VMEM((16 * _LANES,), jnp.int32)]
            + [pltpu.SemaphoreType.DMA for _ in range(2 * _NBUF)]
        ),
        compiler_params=pltpu.CompilerParams(
            use_tc_tiling_on_sc=False, needs_layout_passes=False,
            disable_bounds_checks=True, disable_semaphore_checks=True),
    )
    def embed(idx_hbm, table_hbm, pos_hbm, out_hbm, idx_blk, pos_v, *bufs):
        icol = bufs[:_NBUF]
        rows = bufs[_NBUF:2 * _NBUF]
        outb = bufs[2 * _NBUF:3 * _NBUF]
        btab = bufs[3 * _NBUF]
        otab = bufs[3 * _NBUF + 1]
        gsem = bufs[3 * _NBUF + 2:4 * _NBUF + 2]
        wsem = bufs[4 * _NBUF + 2:]
        wid = lax.axis_index("s") * info.num_cores + lax.axis_index("c")

        pltpu.sync_copy(idx_hbm.at[pl.ds(wid * (BB * S), BB * S)], idx_blk)
        pltpu.sync_copy(pos_hbm, pos_v)

        lane = lax.iota(jnp.int32, _LANES)
        lane_s = lane * S
        oconst = (lane >> 3) * TILE + (lane & 7) * 128

        def tab_body(k, carry):
            t = (lane + k) & 15
            btab[pl.ds(k * _LANES, _LANES)] = t
            otab[pl.ds(k * _LANES, _LANES)] = oconst + t
            return carry

        lax.fori_loop(0, 16, tab_body, 0)

        def stage_gather(s, j):
            for bb in range(BB // _LANES):
                v = plsc.load_gather(
                    idx_blk, [lane_s + (bb * _LANES * S + s)])
                icol[j][pl.ds(bb * _LANES, _LANES)] = v
            pltpu.make_async_copy(
                table_hbm.at[icol[j]], rows[j], gsem[j]).start()

        def write_desc(s, tr, j):
            return pltpu.make_async_copy(
                outb[j].at[pl.ds(tr * TILE, TILE)],
                out_hbm.at[s, tr, wid], wsem[j])

        for j in range(_NBUF - 1):
            stage_gather(j, j)

        def block_body(blk, carry):
            for b in range(_NBUF):
                s = blk * _NBUF + b
                jprev = (b - 1) % _NBUF
                jnext = (b + _NBUF - 1) % _NBUF
                if b == 0:
                    @pl.when(blk >= 1)
                    def _():
                        for tr in range(DT):
                            write_desc(s - 1, tr, jprev).wait()
                else:
                    for tr in range(DT):
                        write_desc(s - 1, tr, jprev).wait()
                @pl.when(s + _NBUF - 1 < S)
                def _():
                    stage_gather(s + _NBUF - 1, jnext)
                pltpu.make_async_copy(
                    table_hbm.at[icol[b]], rows[b], gsem[b]).wait()

                @plsc.parallel_loop(0, 64, unroll=8)
                def _(i):
                    k = i >> 2
                    db = i & 3
                    bt = btab[pl.ds(k * _LANES, _LANES)]
                    ot = otab[pl.ds(k * _LANES, _LANES)]
                    if True:
                        pv = pos_v[s, pl.ds(db * _LANES, _LANES)]
                        dv = lane + db * _LANES
                        for bb in range(BB // _LANES):
                            base_w = db * 2 * TILE + bb * _LANES
                            v = plsc.load_gather(
                                rows[b].at[pl.ds(bb * _LANES, _LANES)],
                                [bt, dv])
                            plsc.store_scatter(
                                outb[b].at[pl.ds(base_w, 1936)],
                                [ot], v + pv)
                for tr in range(DT):
                    write_desc(s, tr, b).start()
            return carry

        lax.fori_loop(0, S // _NBUF, block_body, 0)
        for tr in range(DT):
            write_desc(S - 1, tr, (S - 1) % _NBUF).wait()

    return embed


def kernel(inputs, token_table, position_table):
    B, S = inputs.shape
    V, D = token_table.shape
    NW = 32
    idx_flat = inputs.reshape(B * S).astype(jnp.int32)
    fn = _build_kernel(B, S, D, V)
    out = fn(idx_flat, token_table, position_table)
    out = out.reshape(S, D // 8, NW, 8, 128)
    out = out.transpose(2, 4, 0, 1, 3)
    return out.reshape(B, S, D)


# speedup vs baseline: 1.7641x; 1.0021x over previous
import functools

import jax
import jax.numpy as jnp
from jax import lax
from jax.experimental import pallas as pl
from jax.experimental.pallas import tpu as pltpu
from jax.experimental.pallas import tpu_sc as plsc

_LANES = 16
_NBUF = 4


@functools.cache
def _build_kernel(B, S, D, V):
    info = plsc.get_sparse_core_info()
    NW = info.num_cores * info.num_subcores
    BB = B // NW
    DT = D // 8
    TILE = 8 * 128
    assert BB == 128 and S % _NBUF == 0

    mesh = plsc.VectorSubcoreMesh(core_axis_name="c", subcore_axis_name="s")

    @functools.partial(
        pl.kernel,
        out_type=jax.ShapeDtypeStruct((S, DT, NW, TILE), jnp.float32),
        mesh=mesh,
        scratch_types=(
            [pltpu.VMEM((BB * S,), jnp.int32)]
            + [pltpu.VMEM((S, D), jnp.float32)]
            + [pltpu.VMEM((BB,), jnp.int32) for _ in range(_NBUF)]
            + [pltpu.VMEM((BB, D), jnp.float32) for _ in range(_NBUF)]
            + [pltpu.VMEM((DT * TILE,), jnp.float32) for _ in range(_NBUF)]
            + [pltpu.VMEM((16 * _LANES,), jnp.int32)]
            + [pltpu.VMEM((16 * _LANES,), jnp.int32)]
            + [pltpu.SemaphoreType.DMA for _ in range(2 * _NBUF)]
        ),
        compiler_params=pltpu.CompilerParams(
            use_tc_tiling_on_sc=False, needs_layout_passes=False,
            disable_bounds_checks=True, disable_semaphore_checks=True),
    )
    def embed(idx_hbm, table_hbm, pos_hbm, out_hbm, idx_blk, pos_v, *bufs):
        icol = bufs[:_NBUF]
        rows = bufs[_NBUF:2 * _NBUF]
        outb = bufs[2 * _NBUF:3 * _NBUF]
        btab = bufs[3 * _NBUF]
        otab = bufs[3 * _NBUF + 1]
        gsem = bufs[3 * _NBUF + 2:4 * _NBUF + 2]
        wsem = bufs[4 * _NBUF + 2:]
        wid = lax.axis_index("s") * info.num_cores + lax.axis_index("c")

        pltpu.sync_copy(idx_hbm.at[pl.ds(wid * (BB * S), BB * S)], idx_blk)
        pltpu.sync_copy(pos_hbm, pos_v)

        lane = lax.iota(jnp.int32, _LANES)
        lane_s = lane * S
        oconst = (lane >> 3) * TILE + (lane & 7) * 128

        def tab_body(k, carry):
            t = (lane + k) & 15
            btab[pl.ds(k * _LANES, _LANES)] = t
            otab[pl.ds(k * _LANES, _LANES)] = oconst + t
            return carry

        lax.fori_loop(0, 16, tab_body, 0)

        def stage_gather(s, j):
            for bb in range(BB // _LANES):
                v = plsc.load_gather(
                    idx_blk, [lane_s + (bb * _LANES * S + s)])
                icol[j][pl.ds(bb * _LANES, _LANES)] = v
            pltpu.make_async_copy(
                table_hbm.at[icol[j]], rows[j], gsem[j]).start()

        def write_desc(s, tr, j):
            return pltpu.make_async_copy(
                outb[j].at[pl.ds(tr * TILE, TILE)],
                out_hbm.at[s, tr, wid], wsem[j])

        for j in range(_NBUF - 1):
            stage_gather(j, j)

        def block_body(blk, carry):
            for b in range(_NBUF):
                s = blk * _NBUF + b
                jprev = (b - 1) % _NBUF
                jnext = (b + _NBUF - 1) % _NBUF
                if b == 0:
                    @pl.when(blk >= 1)
                    def _():
                        for tr in range(DT):
                            write_desc(s - 1, tr, jprev).wait()
                else:
                    for tr in range(DT):
                        write_desc(s - 1, tr, jprev).wait()
                @pl.when(s + _NBUF - 1 < S)
                def _():
                    stage_gather(s + _NBUF - 1, jnext)
                pltpu.make_async_copy(
                    table_hbm.at[icol[b]], rows[b], gsem[b]).wait()

                @plsc.parallel_loop(0, 64, unroll=8)
                def _(i):
                    k = i >> 2
                    db = i & 3
                    bt = btab[pl.ds(k * _LANES, _LANES)]
                    ot = otab[pl.ds(k * _LANES, _LANES)]
                    pv = pos_v[s, pl.ds(db * _LANES, _LANES)]
                    dv = lane + db * _LANES
                    for bb in range(BB // _LANES):
                        base_w = db * 2 * TILE + bb * _LANES
                        v = plsc.load_gather(
                            rows[b].at[pl.ds(bb * _LANES, _LANES)],
                            [bt, dv])
                        plsc.store_scatter(
                            outb[b].at[pl.ds(base_w, 1936)], [ot], v + pv)
                for tr in range(DT):
                    write_desc(s, tr, b).start()
            return carry

        lax.fori_loop(0, S // _NBUF, block_body, 0)
        for tr in range(DT):
            write_desc(S - 1, tr, (S - 1) % _NBUF).wait()

    return embed


def kernel(inputs, token_table, position_table):
    B, S = inputs.shape
    V, D = token_table.shape
    NW = 32
    idx_flat = inputs.reshape(B * S).astype(jnp.int32)
    fn = _build_kernel(B, S, D, V)
    out = fn(idx_flat, token_table, position_table)
    out = out.reshape(S, D // 8, NW, 8, 128)
    out = out.transpose(2, 4, 0, 1, 3)
    return out.reshape(B, S, D)
